# baseline (speedup 1.0000x reference)
import functools

import jax
import jax.numpy as jnp
from jax import lax
from jax.experimental import pallas as pl
from jax.experimental.pallas import tpu as pltpu
from jax.experimental.pallas import tpu_sc as plsc

B, S, D = 4, 8192, 4096
L = 16
NS = 16
CHUNK = (B * S) // NS
PER_BATCH = S // CHUNK
UNROLL = 4

_mesh = plsc.VectorSubcoreMesh(
    core_axis_name="c", subcore_axis_name="s", num_cores=1
)


@functools.partial(
    pl.kernel,
    mesh=_mesh,
    out_type=jax.ShapeDtypeStruct((B * B, D), jnp.float32),
    compiler_params=pltpu.CompilerParams(
        needs_layout_passes=False,
        disable_bounds_checks=True,
        disable_semaphore_checks=True,
        skip_device_barrier=True,
    ),
    scratch_types=[
        pltpu.VMEM((CHUNK,), jnp.int32),
        pltpu.VMEM((L,), jnp.int32),
        pltpu.VMEM_SHARED((NS * L,), jnp.int32),
        pltpu.VMEM((PER_BATCH * L,), jnp.int32),
        pltpu.VMEM((L,), jnp.int32),
        pltpu.VMEM((1, D), jnp.float32),
        pltpu.SemaphoreType.DMA,
    ],
)
def _pool(lhs_hbm, mask_hbm, out_hbm,
          chunk_v, pad_v, sums_sh, row_sums_v, idxs_v, row_v, sem):
    s = lax.axis_index("s")

    pltpu.sync_copy(mask_hbm.at[pl.ds(s * CHUNK, CHUNK)], chunk_v)

    def step(i, accs):
        return tuple(
            accs[u]
            + plsc.all_reduce_population_count(
                chunk_v[pl.ds((i * UNROLL + u) * L, L)] != 0
            )
            for u in range(UNROLL)
        )

    zeros = jnp.zeros((L,), jnp.int32)
    accs = lax.fori_loop(0, CHUNK // (L * UNROLL), step, (zeros,) * UNROLL)
    acc = accs[0] + accs[1] + (accs[2] + accs[3])
    pad_v[...] = acc
    pltpu.sync_copy(pad_v, sums_sh.at[pl.ds(s * L, L)])
    plsc.subcore_barrier()

    b = s // B
    j = s % B
    pltpu.sync_copy(
        sums_sh.at[pl.ds(j * (PER_BATCH * L), PER_BATCH * L)], row_sums_v
    )
    v = zeros
    for k in range(PER_BATCH):
        v = v + row_sums_v[pl.ds(k * L, L)]
    v = jnp.where(v < 1, S, v)
    idx = b * S + v - 1
    idxs_v[...] = idx
    pltpu.async_copy(lhs_hbm.at[idxs_v.at[pl.ds(0, 1)]], row_v, sem).wait()
    pltpu.sync_copy(row_v, out_hbm.at[pl.ds(s, 1)])


def kernel(last_hidden_state, attention_mask):
    lhs2 = last_hidden_state.reshape(B * S, D)
    mask = attention_mask.astype(jnp.int32).reshape(B * S)
    out = _pool(lhs2, mask)
    return out.reshape(B, B, D)


# speedup vs baseline: 1.0083x; 1.0083x over previous
import functools

import jax
import jax.numpy as jnp
from jax import lax
from jax.experimental import pallas as pl
from jax.experimental.pallas import tpu as pltpu
from jax.experimental.pallas import tpu_sc as plsc

B, S, D = 4, 8192, 4096
L = 16
NS = 16
CHUNK = (B * S) // NS
PER_BATCH = S // CHUNK
UNROLL = 4

_mesh = plsc.VectorSubcoreMesh(
    core_axis_name="c", subcore_axis_name="s", num_cores=1
)


@functools.partial(
    pl.kernel,
    mesh=_mesh,
    out_type=jax.ShapeDtypeStruct((B * B, D), jnp.float32),
    compiler_params=pltpu.CompilerParams(needs_layout_passes=False),
    scratch_types=[
        pltpu.VMEM((CHUNK,), jnp.int32),
        pltpu.VMEM((L,), jnp.int32),
        pltpu.VMEM_SHARED((NS * L,), jnp.int32),
        pltpu.VMEM((PER_BATCH * L,), jnp.int32),
        pltpu.VMEM((L,), jnp.int32),
        pltpu.VMEM((1, D), jnp.float32),
        pltpu.SemaphoreType.DMA,
    ],
)
def _pool(lhs_hbm, mask_hbm, out_hbm,
          chunk_v, pad_v, sums_sh, row_sums_v, idxs_v, row_v, sem):
    s = lax.axis_index("s")

    pltpu.sync_copy(mask_hbm.at[pl.ds(s * CHUNK, CHUNK)], chunk_v)

    def step(i, accs):
        return tuple(
            accs[u]
            + plsc.all_reduce_population_count(
                chunk_v[pl.ds((i * UNROLL + u) * L, L)] != 0
            )
            for u in range(UNROLL)
        )

    zeros = jnp.zeros((L,), jnp.int32)
    accs = lax.fori_loop(0, CHUNK // (L * UNROLL), step, (zeros,) * UNROLL)
    acc = accs[0] + accs[1] + (accs[2] + accs[3])
    pad_v[...] = acc
    pltpu.sync_copy(pad_v, sums_sh.at[pl.ds(s * L, L)])
    plsc.subcore_barrier()

    b = s // B
    j = s % B
    pltpu.sync_copy(
        sums_sh.at[pl.ds(j * (PER_BATCH * L), PER_BATCH * L)], row_sums_v
    )
    v = zeros
    for k in range(PER_BATCH):
        v = v + row_sums_v[pl.ds(k * L, L)]
    v = jnp.where(v < 1, S, v)
    idx = b * S + v - 1
    idxs_v[...] = idx
    pltpu.async_copy(lhs_hbm.at[idxs_v.at[pl.ds(0, 1)]], row_v, sem).wait()
    pltpu.sync_copy(row_v, out_hbm.at[pl.ds(s, 1)])


def kernel(last_hidden_state, attention_mask):
    lhs2 = last_hidden_state.reshape(B * S, D)
    mask = attention_mask.astype(jnp.int32).reshape(B * S)
    out = _pool(lhs2, mask)
    return out.reshape(B, B, D)
